# baseline (speedup 1.0000x reference)
import functools

import jax
import jax.numpy as jnp
import numpy as np
from jax import lax
from jax.experimental import pallas as pl
from jax.experimental.pallas import tpu as pltpu
from jax.experimental.pallas import tpu_sc as plsc

H_DIM = 32
W_DIM = 32
HW = H_DIM * W_DIM

NC = 2
NS = 16
NW = NC * NS
L = 16


def _diag_perm(mode):
    idx = []
    for d in range(H_DIM + W_DIM - 1):
        for i in range(H_DIM):
            j = d - i if mode == "rd" else i - (H_DIM - 1 - d)
            if 0 <= j < W_DIM:
                idx.append(i * W_DIM + j)
    return np.asarray(idx, dtype=np.int32)


_RD_PERM = _diag_perm("rd")
_LD_PERM = _diag_perm("ld")

T_ROWS = 8


@functools.partial(jax.jit, static_argnums=(3,))
def _diag_scan(xf, rd_idx, ld_idx, rows):
    rows_per_w = rows // NW
    n_steps = rows_per_w // T_ROWS
    blk = T_ROWS * HW
    mesh = plsc.VectorSubcoreMesh(core_axis_name="c", subcore_axis_name="s")

    @functools.partial(
        pl.kernel,
        mesh=mesh,
        compiler_params=pltpu.CompilerParams(needs_layout_passes=False),
        out_type=(
            jax.ShapeDtypeStruct((rows * HW,), jnp.float32),
            jax.ShapeDtypeStruct((rows * HW,), jnp.float32),
        ),
        scratch_types=[
            pltpu.VMEM((HW,), jnp.int32),
            pltpu.VMEM((HW,), jnp.int32),
            pltpu.VMEM((blk,), jnp.float32),
            pltpu.VMEM((blk,), jnp.float32),
            pltpu.VMEM((blk,), jnp.float32),
        ],
    )
    def k(x_hbm, rdi_hbm, ldi_hbm, yrd_hbm, yld_hbm, rd_v, ld_v, in_v, ord_v, old_v):
        wid = lax.axis_index("s") * NC + lax.axis_index("c")
        base = wid * rows_per_w * HW
        pltpu.sync_copy(rdi_hbm, rd_v)
        pltpu.sync_copy(ldi_hbm, ld_v)

        def step(g, carry):
            off = base + g * blk
            pltpu.sync_copy(x_hbm.at[pl.ds(off, blk)], in_v)

            def chunk(j, c2):
                ird = rd_v[pl.ds(j * L, L)]
                ild = ld_v[pl.ds(j * L, L)]
                for t in range(T_ROWS):
                    tb = jnp.int32(t * HW)
                    ord_v[pl.ds(t * HW + j * L, L)] = plsc.load_gather(in_v, [ird + tb])
                    old_v[pl.ds(t * HW + j * L, L)] = plsc.load_gather(in_v, [ild + tb])
                return c2

            lax.fori_loop(0, HW // L, chunk, 0)
            pltpu.sync_copy(ord_v, yrd_hbm.at[pl.ds(off, blk)])
            pltpu.sync_copy(old_v, yld_hbm.at[pl.ds(off, blk)])
            return carry

        lax.fori_loop(0, n_steps, step, 0)

    return k(xf, rd_idx, ld_idx)


def kernel(x):
    B, C, H, W = x.shape
    rows = B * C
    xf = x.reshape(rows * HW)
    yrd, yld = _diag_scan(xf, jnp.asarray(_RD_PERM), jnp.asarray(_LD_PERM), rows)
    return yrd.reshape(B, C, HW), yld.reshape(B, C, HW)


# speedup vs baseline: 1.6821x; 1.6821x over previous
import functools

import jax
import jax.numpy as jnp
import numpy as np
from jax import lax
from jax.experimental import pallas as pl
from jax.experimental.pallas import tpu as pltpu
from jax.experimental.pallas import tpu_sc as plsc

H_DIM = 32
W_DIM = 32
HW = H_DIM * W_DIM

NC = 2
NS = 16
NW = NC * NS
L = 16


def _diag_perm(mode):
    idx = []
    for d in range(H_DIM + W_DIM - 1):
        for i in range(H_DIM):
            j = d - i if mode == "rd" else i - (H_DIM - 1 - d)
            if 0 <= j < W_DIM:
                idx.append(i * W_DIM + j)
    return np.asarray(idx, dtype=np.int32)


_RD_PERM = _diag_perm("rd")
_LD_PERM = _diag_perm("ld")

T_ROWS = 16


@functools.partial(jax.jit, static_argnums=(3,))
def _diag_scan(xf, rd_idx, ld_idx, rows):
    rows_per_w = rows // NW
    blk = T_ROWS * HW
    n_steps = rows_per_w // T_ROWS
    n2 = n_steps // 2
    total = rows * HW
    mesh = plsc.VectorSubcoreMesh(core_axis_name="c", subcore_axis_name="s")

    @functools.partial(
        pl.kernel,
        mesh=mesh,
        compiler_params=pltpu.CompilerParams(needs_layout_passes=False),
        out_type=(
            jax.ShapeDtypeStruct((total,), jnp.float32),
            jax.ShapeDtypeStruct((total,), jnp.float32),
        ),
        scratch_types=[
            pltpu.VMEM((HW,), jnp.int32),
            pltpu.VMEM((HW,), jnp.int32),
            pltpu.VMEM((blk,), jnp.float32),
            pltpu.VMEM((blk,), jnp.float32),
            pltpu.VMEM((blk,), jnp.float32),
            pltpu.VMEM((blk,), jnp.float32),
            pltpu.VMEM((blk,), jnp.float32),
            pltpu.VMEM((blk,), jnp.float32),
            pltpu.SemaphoreType.DMA,
            pltpu.SemaphoreType.DMA,
            pltpu.SemaphoreType.DMA,
            pltpu.SemaphoreType.DMA,
            pltpu.SemaphoreType.DMA,
            pltpu.SemaphoreType.DMA,
        ],
    )
    def k(x_hbm, rdi_hbm, ldi_hbm, yrd_hbm, yld_hbm,
          rd_v, ld_v, in0, in1, or0, or1, ol0, ol1,
          in_s0, in_s1, or_s0, or_s1, ol_s0, ol_s1):
        wid = lax.axis_index("s") * NC + lax.axis_index("c")
        base = wid * rows_per_w * HW
        pltpu.sync_copy(rdi_hbm, rd_v)
        pltpu.sync_copy(ldi_hbm, ld_v)

        def in_off(g):
            return jnp.minimum(base + g * blk, total - blk)

        def permute(in_v, ord_v, old_v):
            @plsc.parallel_loop(0, HW // L)
            def _(j):
                ird = rd_v[pl.ds(j * L, L)]
                ild = ld_v[pl.ds(j * L, L)]
                for t in range(T_ROWS):
                    tb = jnp.int32(t * HW)
                    o = t * HW + j * L
                    ord_v[pl.ds(o, L)] = plsc.load_gather(in_v, [ird + tb])
                    old_v[pl.ds(o, L)] = plsc.load_gather(in_v, [ild + tb])

        def half(i, g, in_v, ord_v, old_v, in_s, or_s, ol_s):
            pltpu.make_async_copy(x_hbm.at[pl.ds(0, blk)], in_v, in_s).wait()

            @pl.when(i > 0)
            def _():
                pltpu.make_async_copy(ord_v, yrd_hbm.at[pl.ds(0, blk)], or_s).wait()
                pltpu.make_async_copy(old_v, yld_hbm.at[pl.ds(0, blk)], ol_s).wait()

            permute(in_v, ord_v, old_v)
            off = base + g * blk
            pltpu.async_copy(ord_v, yrd_hbm.at[pl.ds(off, blk)], or_s)
            pltpu.async_copy(old_v, yld_hbm.at[pl.ds(off, blk)], ol_s)
            pltpu.async_copy(x_hbm.at[pl.ds(in_off(g + 2), blk)], in_v, in_s)

        pltpu.async_copy(x_hbm.at[pl.ds(in_off(0), blk)], in0, in_s0)
        pltpu.async_copy(x_hbm.at[pl.ds(in_off(1), blk)], in1, in_s1)

        def body(i, carry):
            half(i, 2 * i, in0, or0, ol0, in_s0, or_s0, ol_s0)
            half(i, 2 * i + 1, in1, or1, ol1, in_s1, or_s1, ol_s1)
            return carry

        lax.fori_loop(0, n2, body, 0)

        pltpu.make_async_copy(x_hbm.at[pl.ds(0, blk)], in0, in_s0).wait()
        pltpu.make_async_copy(x_hbm.at[pl.ds(0, blk)], in1, in_s1).wait()
        pltpu.make_async_copy(or0, yrd_hbm.at[pl.ds(0, blk)], or_s0).wait()
        pltpu.make_async_copy(or1, yrd_hbm.at[pl.ds(0, blk)], or_s1).wait()
        pltpu.make_async_copy(ol0, yld_hbm.at[pl.ds(0, blk)], ol_s0).wait()
        pltpu.make_async_copy(ol1, yld_hbm.at[pl.ds(0, blk)], ol_s1).wait()

    return k(xf, rd_idx, ld_idx)


def kernel(x):
    B, C, H, W = x.shape
    rows = B * C
    xf = x.reshape(rows * HW)
    yrd, yld = _diag_scan(xf, jnp.asarray(_RD_PERM), jnp.asarray(_LD_PERM), rows)
    return yrd.reshape(B, C, HW), yld.reshape(B, C, HW)
